# Initial kernel scaffold; baseline (speedup 1.0000x reference)
import jax
import jax.numpy as jnp
from jax.experimental import pallas as pl


def kernel(x, orig_len):
    raise NotImplementedError("write your pallas kernel here")


# speedup vs baseline: 1.5810x; 1.5810x over previous
import jax
import jax.numpy as jnp
from jax.experimental import pallas as pl
from jax.experimental.pallas import tpu as pltpu

NUM_LEVELS = 3
SHIFT = -16
NEG_INF = float("-inf")


def _window_bounds(leff):
    bounds = [(jnp.int32(0), leff)]
    for lvl in (1, 2):
        d = 2 ** lvl
        k = (leff + d - 1) // d
        s = leff // d
        for j in range(d):
            lo = jnp.int32(j) * s
            hi = jnp.minimum(lo + k, leff)
            bounds.append((lo, hi))
    return bounds


def _tc_body(lens_ref, x_ref, o1_ref, o2_ref, o3_ref):
    i = pl.program_id(0)
    t = pl.program_id(1)
    T = x_ref.shape[2]
    L = pl.num_programs(1) * T

    leff = jnp.minimum(lens_ref[i] - SHIFT, L)
    xb = x_ref[0]
    pos = jax.lax.broadcasted_iota(jnp.int32, (1, T), 1) + t * T

    @pl.when(t == 0)
    def _():
        o1_ref[...] = jnp.full_like(o1_ref, NEG_INF)
        o2_ref[...] = jnp.full_like(o2_ref, NEG_INF)
        o3_ref[...] = jnp.full_like(o3_ref, NEG_INF)

    maxes = []
    for lo, hi in _window_bounds(leff):
        mask = (pos >= lo) & (pos < hi)
        maxes.append(jnp.max(jnp.where(mask, xb, NEG_INF), axis=1))

    o1_ref[0, 0, :] = jnp.maximum(o1_ref[0, 0, :], maxes[0])
    for j in range(2):
        o2_ref[0, :, j] = jnp.maximum(o2_ref[0, :, j], maxes[1 + j])
    for j in range(4):
        o3_ref[0, :, j] = jnp.maximum(o3_ref[0, :, j], maxes[3 + j])


def kernel(x, orig_len):
    B, C, L = x.shape
    T = 1024
    NT = L // T
    lens = jnp.asarray(orig_len, jnp.int32)

    grid_spec = pltpu.PrefetchScalarGridSpec(
        num_scalar_prefetch=1,
        grid=(B, NT),
        in_specs=[
            pl.BlockSpec((1, C, T), lambda i, t, lens: (i, 0, t)),
        ],
        out_specs=[
            pl.BlockSpec((1, 1, C), lambda i, t, lens: (i, 0, 0)),
            pl.BlockSpec((1, C, 2), lambda i, t, lens: (i, 0, 0)),
            pl.BlockSpec((1, C, 4), lambda i, t, lens: (i, 0, 0)),
        ],
    )
    o1, o2, o3 = pl.pallas_call(
        _tc_body,
        grid_spec=grid_spec,
        out_shape=[
            jax.ShapeDtypeStruct((B, 1, C), jnp.float32),
            jax.ShapeDtypeStruct((B, C, 2), jnp.float32),
            jax.ShapeDtypeStruct((B, C, 4), jnp.float32),
        ],
    )(lens, x)
    return jnp.concatenate(
        [o1.reshape(B, C), o2.reshape(B, 2 * C), o3.reshape(B, 4 * C)], axis=1
    )
